# baseline (speedup 1.0000x reference)
import functools
import jax
import jax.numpy as jnp
from jax import lax
from jax.experimental import pallas as pl
from jax.experimental.pallas import tpu as pltpu
from jax.experimental.pallas import tpu_sc as plsc

N = 2048
NFEAT = 128
NHID = 64
NC, NS, L = 2, 16, 16
NW = NC * NS
RW = N // NW
CH = 16
NCH = RW // CH


def _prep_kernel(x_ref, Wl_ref, b_ref, a1_ref, a2r_ref,
                 h_ref, u_ref, v_ref):
    x = x_ref[...]
    Wl = Wl_ref[...]
    h = jax.lax.dot_general(
        x, Wl, (((1,), (1,)), ((), ())),
        preferred_element_type=jnp.float32) + b_ref[...]
    h_ref[...] = h
    u_ref[...] = jnp.dot(h, a1_ref[...], preferred_element_type=jnp.float32)
    wv = jnp.dot(a2r_ref[...], Wl, preferred_element_type=jnp.float32)
    cv = jnp.sum(a2r_ref[...] * b_ref[...])
    v_ref[...] = jax.lax.dot_general(
        wv, x, (((1,), (1,)), ((), ())),
        preferred_element_type=jnp.float32) + cv


def _sc_edge_body(adj_ref, u_ref, v_ref, out_ref, uv, vv, v2v, av, ov):
    c = lax.axis_index("c")
    s = lax.axis_index("s")
    wid = s * NC + c
    base = wid * RW
    pltpu.sync_copy(u_ref.at[pl.ds(base, RW)], uv)
    pltpu.sync_copy(v_ref, vv)

    def dbl(j, carry):
        v2v[pl.ds(j * L, L)] = vv[pl.ds(j * L, L)] * 2.0
        return carry

    lax.fori_loop(0, N // L, dbl, 0)

    for k in range(NCH):
        r0 = base + k * CH
        pltpu.sync_copy(adj_ref.at[pl.ds(r0, CH)], av)
        u16 = uv[pl.ds(k * CH, L)] * 2.0
        for i in range(CH):
            tu = u16[i]

            def col(j, carry2, i=i, tu=tu):
                z = v2v[pl.ds(j * L, L)] + tu
                e = jnp.exp(z)
                t = 1.0 - 2.0 / (e + 1.0)
                ov[i, pl.ds(j * L, L)] = av[i, pl.ds(j * L, L)] * t
                return carry2

            lax.fori_loop(0, N // L, col, 0)
        pltpu.sync_copy(ov, out_ref.at[pl.ds(r0, CH)])


def _sc_edge(adj, u, v, interpret=False):
    fn = functools.partial(
        pl.kernel,
        out_type=jax.ShapeDtypeStruct((N, N), jnp.float32),
        mesh=plsc.VectorSubcoreMesh(core_axis_name="c", subcore_axis_name="s",
                                    num_cores=NC, num_subcores=NS),
        scratch_types=[
            pltpu.VMEM((RW,), jnp.float32),
            pltpu.VMEM((N,), jnp.float32),
            pltpu.VMEM((N,), jnp.float32),
            pltpu.VMEM((CH, N), jnp.float32),
            pltpu.VMEM((CH, N), jnp.float32),
        ],
        interpret=interpret,
    )(_sc_edge_body)
    return fn(adj, u, v)


def kernel(adj, x, W_lin, b_lin, W_att, _interpret=False):
    b_row = b_lin.reshape(1, NHID)
    a1_col = W_att[0, :NHID].reshape(NHID, 1)
    a2_row = W_att[:, NHID:]

    h, u, v = pl.pallas_call(
        _prep_kernel,
        out_shape=(
            jax.ShapeDtypeStruct((N, NHID), jnp.float32),
            jax.ShapeDtypeStruct((N, 1), jnp.float32),
            jax.ShapeDtypeStruct((1, N), jnp.float32),
        ),
        interpret=_interpret,
    )(x, W_lin, b_row, a1_col, a2_row)

    new_adj = _sc_edge(adj, u.reshape(N), v.reshape(N),
                       interpret=_interpret)
    return (new_adj, h)


# speedup vs baseline: 15.1083x; 15.1083x over previous
import jax
import jax.numpy as jnp
from jax.experimental import pallas as pl
from jax.experimental.pallas import tpu as pltpu

N = 2048
NFEAT = 128
NHID = 64
ROWS = 1024


def _fused_kernel(x_ref, Wl_ref, b_ref, Wa_ref, adj_ref,
                  out_ref, h_ref, u_scr, v_scr):
    i = pl.program_id(0)

    @pl.when(i == 0)
    def _prep():
        x = x_ref[...]
        Wl = Wl_ref[...]
        h = jax.lax.dot_general(
            x, Wl, (((1,), (1,)), ((), ())),
            preferred_element_type=jnp.float32) + b_ref[...]
        h_ref[...] = h
        u_scr[...] = jax.lax.dot_general(
            h, Wa_ref[:, :NHID], (((1,), (1,)), ((), ())),
            preferred_element_type=jnp.float32)
        a2r = Wa_ref[:, NHID:]
        wv = jnp.dot(a2r, Wl,
                     preferred_element_type=jnp.float32)
        cv = jnp.sum(a2r * b_ref[...])
        v_scr[...] = jax.lax.dot_general(
            wv, x, (((1,), (1,)), ((), ())),
            preferred_element_type=jnp.float32) + cv

    u = u_scr[pl.ds(i * ROWS, ROWS), :]
    out_ref[...] = adj_ref[...] * jnp.tanh(u + v_scr[...])


def kernel(adj, x, W_lin, b_lin, W_att):
    b_row = b_lin.reshape(1, NHID)

    new_adj, h = pl.pallas_call(
        _fused_kernel,
        grid=(N // ROWS,),
        in_specs=[
            pl.BlockSpec((N, NFEAT), lambda i: (0, 0)),
            pl.BlockSpec((NHID, NFEAT), lambda i: (0, 0)),
            pl.BlockSpec((1, NHID), lambda i: (0, 0)),
            pl.BlockSpec((1, 2 * NHID), lambda i: (0, 0)),
            pl.BlockSpec((ROWS, N), lambda i: (i, 0)),
        ],
        out_specs=[
            pl.BlockSpec((ROWS, N), lambda i: (i, 0)),
            pl.BlockSpec((N, NHID), lambda i: (0, 0)),
        ],
        out_shape=[
            jax.ShapeDtypeStruct((N, N), jnp.float32),
            jax.ShapeDtypeStruct((N, NHID), jnp.float32),
        ],
        scratch_shapes=[
            pltpu.VMEM((N, 1), jnp.float32),
            pltpu.VMEM((1, N), jnp.float32),
        ],
    )(x, W_lin, b_row, W_att, adj)

    return (new_adj, h)


# speedup vs baseline: 15.1112x; 1.0002x over previous
import jax
import jax.numpy as jnp
from jax.experimental import pallas as pl
from jax.experimental.pallas import tpu as pltpu

N = 2048
NFEAT = 128
NHID = 64
ROWS = 1024


def _fused_kernel(x_ref, Wl_ref, b_ref, Wa_ref, adj_ref,
                  out_ref, h_ref, u_scr, v_scr):
    i = pl.program_id(0)

    @pl.when(i == 0)
    def _prep():
        x = x_ref[...]
        Wl = Wl_ref[...]
        h = jax.lax.dot_general(
            x, Wl, (((1,), (1,)), ((), ())),
            preferred_element_type=jnp.float32) + b_ref[...]
        h_ref[...] = h
        u_scr[...] = jax.lax.dot_general(
            h, Wa_ref[:, :NHID], (((1,), (1,)), ((), ())),
            preferred_element_type=jnp.float32)
        a2r = Wa_ref[:, NHID:]
        wv = jnp.dot(a2r, Wl,
                     preferred_element_type=jnp.float32)
        cv = jnp.sum(a2r * b_ref[...])
        v_scr[...] = jax.lax.dot_general(
            wv, x, (((1,), (1,)), ((), ())),
            preferred_element_type=jnp.float32) + cv

    u = u_scr[pl.ds(i * ROWS, ROWS), :]
    out_ref[...] = adj_ref[...] * jnp.tanh(u + v_scr[...])


def kernel(adj, x, W_lin, b_lin, W_att):

    new_adj, h = pl.pallas_call(
        _fused_kernel,
        grid=(N // ROWS,),
        in_specs=[
            pl.BlockSpec((N, NFEAT), lambda i: (0, 0)),
            pl.BlockSpec((NHID, NFEAT), lambda i: (0, 0)),
            pl.BlockSpec((NHID,), lambda i: (0,)),
            pl.BlockSpec((1, 2 * NHID), lambda i: (0, 0)),
            pl.BlockSpec((ROWS, N), lambda i: (i, 0)),
        ],
        out_specs=[
            pl.BlockSpec((ROWS, N), lambda i: (i, 0)),
            pl.BlockSpec((N, NHID), lambda i: (0, 0)),
        ],
        out_shape=[
            jax.ShapeDtypeStruct((N, N), jnp.float32),
            jax.ShapeDtypeStruct((N, NHID), jnp.float32),
        ],
        scratch_shapes=[
            pltpu.VMEM((N, 1), jnp.float32),
            pltpu.VMEM((1, N), jnp.float32),
        ],
    )(x, W_lin, b_lin, W_att, adj)

    return (new_adj, h)
